# baseline (speedup 1.0000x reference)
import functools
import math

import jax
import jax.numpy as jnp
from jax import lax
from jax.experimental import pallas as pl
from jax.experimental.pallas import tpu as pltpu
from jax.experimental.pallas import tpu_sc as plsc

_NT = 65536
_C = 32
_NCODES = 1024
_ATTN = 32
_R = 4096
_SCALE = 1.0 / math.sqrt(_ATTN)


def _tc_idx_body(hsT_ref, cb_ref, cbT_ref, wqT_ref, bqT_ref, wk_ref, bk_ref,
                 wvT_ref, bvT_ref, idx_ref, valT_ref, keybf_ref):
    i = pl.program_id(0)

    @pl.when(i == 0)
    def _init():
        key = lax.dot_general(
            cb_ref[...], wk_ref[...], (((1,), (0,)), ((), ()))
        ) + bk_ref[...]
        keybf_ref[...] = key.astype(jnp.bfloat16)
        valT_ref[...] = lax.dot_general(
            wvT_ref[...], cbT_ref[...], (((1,), (0,)), ((), ()))
        ) + bvT_ref[...]

    qT = lax.dot_general(
        wqT_ref[...], hsT_ref[...], (((1,), (0,)), ((), ()))
    ) + bqT_ref[...]
    qT_bf = qT.astype(jnp.bfloat16)
    logitsT = lax.dot_general(
        keybf_ref[...], qT_bf, (((1,), (0,)), ((), ())),
        preferred_element_type=jnp.float32,
    )
    m = jnp.max(logitsT, axis=0, keepdims=True)
    iota = lax.broadcasted_iota(jnp.int32, (_NCODES, _R), 0)
    idx_ref[...] = jnp.min(jnp.where(logitsT == m, iota, _NCODES), axis=0)


_tc_idx_call = pl.pallas_call(
    _tc_idx_body,
    grid=(_NT // _R,),
    in_specs=[
        pl.BlockSpec((_C, _R), lambda i: (0, i)),
        pl.BlockSpec((_NCODES, _C), lambda i: (0, 0)),
        pl.BlockSpec((_C, _NCODES), lambda i: (0, 0)),
        pl.BlockSpec((_C, _ATTN), lambda i: (0, 0)),
        pl.BlockSpec((_ATTN, 1), lambda i: (0, 0)),
        pl.BlockSpec((_C, _ATTN), lambda i: (0, 0)),
        pl.BlockSpec((1, _ATTN), lambda i: (0, 0)),
        pl.BlockSpec((_C, _C), lambda i: (0, 0)),
        pl.BlockSpec((_C, 1), lambda i: (0, 0)),
    ],
    out_specs=[
        pl.BlockSpec((_R,), lambda i: (i,)),
        pl.BlockSpec((_C, _NCODES), lambda i: (0, 0)),
        pl.BlockSpec((_NCODES, _ATTN), lambda i: (0, 0)),
    ],
    out_shape=[
        jax.ShapeDtypeStruct((_NT,), jnp.int32),
        jax.ShapeDtypeStruct((_C, _NCODES), jnp.float32),
        jax.ShapeDtypeStruct((_NCODES, _ATTN), jnp.bfloat16),
    ],
)


def _tc_store_body(hsT_ref, wqT_ref, bqT_ref, keybf_ref, logits_ref):
    qT = lax.dot_general(
        wqT_ref[...], hsT_ref[...], (((1,), (0,)), ((), ()))
    ) + bqT_ref[...]
    logits_ref[...] = lax.dot_general(
        qT.astype(jnp.bfloat16), keybf_ref[...], (((0,), (1,)), ((), ())),
        preferred_element_type=jnp.float32,
    ) * _SCALE


_tc_store_call = pl.pallas_call(
    _tc_store_body,
    grid=(_NT // _R,),
    in_specs=[
        pl.BlockSpec((_C, _R), lambda i: (0, i)),
        pl.BlockSpec((_C, _ATTN), lambda i: (0, 0)),
        pl.BlockSpec((_ATTN, 1), lambda i: (0, 0)),
        pl.BlockSpec((_NCODES, _ATTN), lambda i: (0, 0)),
    ],
    out_specs=[
        pl.BlockSpec((_R, _NCODES), lambda i: (i, 0)),
    ],
    out_shape=[
        jax.ShapeDtypeStruct((_NT, _NCODES), jnp.float32),
    ],
    compiler_params=pltpu.CompilerParams(fuse_transposed_lhs_in_matmul=True),
)


_NW = 32
_BPW = _NT // _NW
_L = 16
_NGRP = _BPW // _L


@functools.lru_cache(maxsize=None)
def _make_sc_gather():
    mesh = plsc.VectorSubcoreMesh(core_axis_name="c", subcore_axis_name="s")

    @functools.partial(
        pl.kernel,
        mesh=mesh,
        compiler_params=pltpu.CompilerParams(use_tc_tiling_on_sc=False,
                                             needs_layout_passes=False),
        out_type=(
            jax.ShapeDtypeStruct((_C * _NT,), jnp.float32),
            jax.ShapeDtypeStruct((_C * _NT,), jnp.float32),
        ),
        scratch_types=[
            pltpu.VMEM((_BPW,), jnp.int32),
            pltpu.VMEM((_C * _NCODES,), jnp.float32),
            pltpu.VMEM((_C * _BPW,), jnp.float32),
            pltpu.SemaphoreType.DMA,
        ],
    )
    def _sc_gather(idx_hbm, tabT_hbm, zq_hbm, zq2_hbm, idx_v, tab_v, out_v,
                   sem):
        wid = lax.axis_index("s") * 2 + lax.axis_index("c")
        base = wid * _BPW
        pltpu.sync_copy(idx_hbm.at[pl.ds(base, _BPW)], idx_v)
        pltpu.sync_copy(tabT_hbm, tab_v)

        def body(g, _):
            off = pl.multiple_of(g * _L, _L)
            idx16 = idx_v[pl.ds(off, _L)]
            tloc = pl.multiple_of((g // 8) * 1024 + (g % 8) * _L, _L)
            for c in range(_C):
                flat = idx16 + (c * _NCODES)
                vals = plsc.load_gather(tab_v, [flat])
                out_v[pl.ds((c // 8) * 16384 + (c % 8) * 128 + tloc, _L)] = vals
            return 0

        lax.fori_loop(0, _NGRP, body, 0)
        copies = []
        ntile = _BPW // 128
        run = ntile * 1024
        for tr in range(_C // 8):
            src = out_v.at[pl.ds(tr * run, run)]
            goff = (tr * (_NT // 128) + base // 128) * 1024
            copies.append(pltpu.async_copy(
                src, zq_hbm.at[pl.ds(goff, run)], sem))
            copies.append(pltpu.async_copy(
                src, zq2_hbm.at[pl.ds(goff, run)], sem))
        for cp in copies:
            cp.wait()

    return _sc_gather


def kernel(hidden_states, codebook_hidden_states, Wq, bq, Wk, bk, Wv, bv):
    hsT = hidden_states.T
    wqT = Wq.T
    bqT = bq.reshape(-1, 1)
    idx1d, valT, keybf = _tc_idx_call(
        hsT, codebook_hidden_states, codebook_hidden_states.T,
        wqT, bqT, Wk, bk.reshape(1, -1), Wv.T, bv.reshape(-1, 1),
    )
    zq_flat, zq2_flat = _make_sc_gather()(idx1d, valT.reshape(-1))
    logits = _tc_store_call(hsT, wqT, bqT, keybf)[0]

    def _untile(flat):
        zqT = flat.reshape(4, _NT // 128, 8, 128).transpose(0, 2, 1, 3)
        return zqT.reshape(_C, _NT).T

    return (logits, idx1d.reshape(-1, 1), _untile(zq_flat), _untile(zq2_flat))


# speedup vs baseline: 1.1460x; 1.1460x over previous
import functools
import math

import jax
import jax.numpy as jnp
from jax import lax
from jax.experimental import pallas as pl
from jax.experimental.pallas import tpu as pltpu
from jax.experimental.pallas import tpu_sc as plsc

_NT = 65536
_C = 32
_NCODES = 1024
_ATTN = 32
_R = 4096
_SCALE = 1.0 / math.sqrt(_ATTN)


def _tc_body(hsT_ref, cb_ref, cbT_ref, wqT_ref, bqT_ref, wk_ref, bk_ref,
             wvT_ref, bvT_ref, logits_ref, idx_ref, valT_ref, keybf_scr):
    i = pl.program_id(0)

    @pl.when(i == 0)
    def _init():
        key = lax.dot_general(
            cb_ref[...], wk_ref[...], (((1,), (0,)), ((), ()))
        ) + bk_ref[...]
        keybf_scr[...] = key.astype(jnp.bfloat16)
        valT_ref[...] = lax.dot_general(
            wvT_ref[...], cbT_ref[...], (((1,), (0,)), ((), ()))
        ) + bvT_ref[...]

    qT = lax.dot_general(
        wqT_ref[...], hsT_ref[...], (((1,), (0,)), ((), ()))
    ) + bqT_ref[...]
    qT_bf = qT.astype(jnp.bfloat16)
    logits = lax.dot_general(
        qT_bf, keybf_scr[...], (((0,), (1,)), ((), ())),
        preferred_element_type=jnp.float32,
    ) * _SCALE
    logits_ref[...] = logits
    logitsT = lax.dot_general(
        keybf_scr[...], qT_bf, (((1,), (0,)), ((), ())),
        preferred_element_type=jnp.float32,
    )
    m = jnp.max(logitsT, axis=0, keepdims=True)
    iota = lax.broadcasted_iota(jnp.int32, (_NCODES, _R), 0)
    idx_ref[...] = jnp.min(jnp.where(logitsT == m, iota, _NCODES), axis=0)


_tc_call = pl.pallas_call(
    _tc_body,
    grid=(_NT // _R,),
    in_specs=[
        pl.BlockSpec((_C, _R), lambda i: (0, i)),
        pl.BlockSpec((_NCODES, _C), lambda i: (0, 0)),
        pl.BlockSpec((_C, _NCODES), lambda i: (0, 0)),
        pl.BlockSpec((_C, _ATTN), lambda i: (0, 0)),
        pl.BlockSpec((_ATTN, 1), lambda i: (0, 0)),
        pl.BlockSpec((_C, _ATTN), lambda i: (0, 0)),
        pl.BlockSpec((1, _ATTN), lambda i: (0, 0)),
        pl.BlockSpec((_C, _C), lambda i: (0, 0)),
        pl.BlockSpec((_C, 1), lambda i: (0, 0)),
    ],
    out_specs=[
        pl.BlockSpec((_R, _NCODES), lambda i: (i, 0)),
        pl.BlockSpec((_R,), lambda i: (i,)),
        pl.BlockSpec((_C, _NCODES), lambda i: (0, 0)),
    ],
    out_shape=[
        jax.ShapeDtypeStruct((_NT, _NCODES), jnp.float32),
        jax.ShapeDtypeStruct((_NT,), jnp.int32),
        jax.ShapeDtypeStruct((_C, _NCODES), jnp.float32),
    ],
    scratch_shapes=[pltpu.VMEM((_NCODES, _ATTN), jnp.bfloat16)],
    compiler_params=pltpu.CompilerParams(fuse_transposed_lhs_in_matmul=True),
)


_NW = 32
_BPW = _NT // _NW
_L = 16
_NGRP = _BPW // _L


@functools.lru_cache(maxsize=None)
def _make_sc_gather():
    mesh = plsc.VectorSubcoreMesh(core_axis_name="c", subcore_axis_name="s")

    @functools.partial(
        pl.kernel,
        mesh=mesh,
        compiler_params=pltpu.CompilerParams(use_tc_tiling_on_sc=False,
                                             needs_layout_passes=False),
        out_type=(
            jax.ShapeDtypeStruct((_C * _NT,), jnp.float32),
            jax.ShapeDtypeStruct((_C * _NT,), jnp.float32),
        ),
        scratch_types=[
            pltpu.VMEM((_BPW,), jnp.int32),
            pltpu.VMEM((_C * _NCODES,), jnp.float32),
            pltpu.VMEM((_C * _BPW,), jnp.float32),
            pltpu.SemaphoreType.DMA,
        ],
    )
    def _sc_gather(idx_hbm, tabT_hbm, zq_hbm, zq2_hbm, idx_v, tab_v, out_v,
                   sem):
        wid = lax.axis_index("s") * 2 + lax.axis_index("c")
        base = wid * _BPW
        pltpu.sync_copy(idx_hbm.at[pl.ds(base, _BPW)], idx_v)
        pltpu.sync_copy(tabT_hbm, tab_v)

        def body(g, _):
            off = pl.multiple_of(g * _L, _L)
            idx16 = idx_v[pl.ds(off, _L)]
            tloc = pl.multiple_of((g // 8) * 1024 + (g % 8) * _L, _L)
            for c in range(_C):
                flat = idx16 + (c * _NCODES)
                vals = plsc.load_gather(tab_v, [flat])
                out_v[pl.ds((c // 8) * 16384 + (c % 8) * 128 + tloc, _L)] = vals
            return 0

        lax.fori_loop(0, _NGRP, body, 0)
        copies = []
        ntile = _BPW // 128
        run = ntile * 1024
        for tr in range(_C // 8):
            src = out_v.at[pl.ds(tr * run, run)]
            goff = (tr * (_NT // 128) + base // 128) * 1024
            copies.append(pltpu.async_copy(
                src, zq_hbm.at[pl.ds(goff, run)], sem))
            copies.append(pltpu.async_copy(
                src, zq2_hbm.at[pl.ds(goff, run)], sem))
        for cp in copies:
            cp.wait()

    return _sc_gather


def kernel(hidden_states, codebook_hidden_states, Wq, bq, Wk, bk, Wv, bv):
    logits, idx1d, valT = _tc_call(
        hidden_states.T, codebook_hidden_states, codebook_hidden_states.T,
        Wq.T, bq.reshape(-1, 1), Wk, bk.reshape(1, -1),
        Wv.T, bv.reshape(-1, 1),
    )
    zq_flat, zq2_flat = _make_sc_gather()(idx1d, valT.reshape(-1))

    def _untile(flat):
        zqT = flat.reshape(4, _NT // 128, 8, 128).transpose(0, 2, 1, 3)
        return zqT.reshape(_C, _NT).T

    return (logits, idx1d.reshape(-1, 1), _untile(zq_flat), _untile(zq2_flat))


# speedup vs baseline: 1.2245x; 1.0685x over previous
import functools
import math

import jax
import jax.numpy as jnp
from jax import lax
from jax.experimental import pallas as pl
from jax.experimental.pallas import tpu as pltpu
from jax.experimental.pallas import tpu_sc as plsc

_NT = 65536
_C = 32
_NCODES = 1024
_ATTN = 32
_R = 4096
_SCALE = 1.0 / math.sqrt(_ATTN)


def _tc_body(hsT_ref, cb_ref, cbT_ref, wqT_ref, bqT_ref, wk_ref, bk_ref,
             wvT_ref, bvT_ref, logits_ref, idx_ref, valT_ref, keybf_scr):
    i = pl.program_id(0)

    @pl.when(i == 0)
    def _init():
        key = lax.dot_general(
            cb_ref[...], wk_ref[...], (((1,), (0,)), ((), ()))
        ) + bk_ref[...]
        keybf_scr[...] = key.astype(jnp.bfloat16)
        valT_ref[...] = lax.dot_general(
            wvT_ref[...], cbT_ref[...], (((1,), (0,)), ((), ()))
        ) + bvT_ref[...]

    qT = lax.dot_general(
        wqT_ref[...], hsT_ref[...], (((1,), (0,)), ((), ()))
    ) + bqT_ref[...]
    qT_bf = qT.astype(jnp.bfloat16)
    logits = lax.dot_general(
        qT_bf, keybf_scr[...], (((0,), (1,)), ((), ())),
        preferred_element_type=jnp.float32,
    ) * _SCALE
    logits_ref[...] = logits
    logitsT = lax.dot_general(
        keybf_scr[...], qT_bf, (((1,), (0,)), ((), ())),
        preferred_element_type=jnp.float32,
    )
    m = jnp.max(logitsT, axis=0, keepdims=True)
    iota = lax.broadcasted_iota(jnp.int32, (_NCODES, _R), 0)
    idx_ref[...] = jnp.min(jnp.where(logitsT == m, iota, _NCODES), axis=0)


_tc_call = pl.pallas_call(
    _tc_body,
    grid=(_NT // _R,),
    in_specs=[
        pl.BlockSpec((_C, _R), lambda i: (0, i)),
        pl.BlockSpec((_NCODES, _C), lambda i: (0, 0)),
        pl.BlockSpec((_C, _NCODES), lambda i: (0, 0)),
        pl.BlockSpec((_C, _ATTN), lambda i: (0, 0)),
        pl.BlockSpec((_ATTN, 1), lambda i: (0, 0)),
        pl.BlockSpec((_C, _ATTN), lambda i: (0, 0)),
        pl.BlockSpec((1, _ATTN), lambda i: (0, 0)),
        pl.BlockSpec((_C, _C), lambda i: (0, 0)),
        pl.BlockSpec((_C, 1), lambda i: (0, 0)),
    ],
    out_specs=[
        pl.BlockSpec((_R, _NCODES), lambda i: (i, 0)),
        pl.BlockSpec((_R,), lambda i: (i,)),
        pl.BlockSpec((_C, _NCODES), lambda i: (0, 0)),
    ],
    out_shape=[
        jax.ShapeDtypeStruct((_NT, _NCODES), jnp.float32),
        jax.ShapeDtypeStruct((_NT,), jnp.int32),
        jax.ShapeDtypeStruct((_C, _NCODES), jnp.float32),
    ],
    scratch_shapes=[pltpu.VMEM((_NCODES, _ATTN), jnp.bfloat16)],
    compiler_params=pltpu.CompilerParams(fuse_transposed_lhs_in_matmul=True),
)


_NW = 32
_BPW = _NT // _NW
_L = 16
_NGRP = _BPW // _L


@functools.lru_cache(maxsize=None)
def _make_sc_gather():
    mesh = plsc.VectorSubcoreMesh(core_axis_name="c", subcore_axis_name="s")

    @functools.partial(
        pl.kernel,
        mesh=mesh,
        compiler_params=pltpu.CompilerParams(use_tc_tiling_on_sc=False,
                                             needs_layout_passes=False),
        out_type=(
            jax.ShapeDtypeStruct((_C * _NT,), jnp.float32),
            jax.ShapeDtypeStruct((_C * _NT,), jnp.float32),
        ),
        scratch_types=[
            pltpu.VMEM((_BPW,), jnp.int32),
            pltpu.VMEM((_C * _NCODES,), jnp.float32),
            pltpu.VMEM((_C * _BPW,), jnp.float32),
            pltpu.SemaphoreType.DMA,
        ],
    )
    def _sc_gather(idx_hbm, tabT_hbm, zq_hbm, zq2_hbm, idx_v, tab_v, out_v,
                   sem):
        wid = lax.axis_index("s") * 2 + lax.axis_index("c")
        base = wid * _BPW
        pltpu.sync_copy(idx_hbm.at[pl.ds(base, _BPW)], idx_v)
        pltpu.sync_copy(tabT_hbm, tab_v)

        @plsc.parallel_loop(0, _NGRP, unroll=2)
        def body(g):
            off = pl.multiple_of(g * _L, _L)
            idx16 = idx_v[pl.ds(off, _L)]
            tloc = pl.multiple_of((g // 8) * 1024 + (g % 8) * _L, _L)
            for c in range(_C):
                flat = idx16 + (c * _NCODES)
                vals = plsc.load_gather(tab_v, [flat])
                out_v[pl.ds((c // 8) * 16384 + (c % 8) * 128 + tloc, _L)] = vals
        copies = []
        ntile = _BPW // 128
        run = ntile * 1024
        for tr in range(_C // 8):
            src = out_v.at[pl.ds(tr * run, run)]
            goff = (tr * (_NT // 128) + base // 128) * 1024
            copies.append(pltpu.async_copy(
                src, zq_hbm.at[pl.ds(goff, run)], sem))
            copies.append(pltpu.async_copy(
                src, zq2_hbm.at[pl.ds(goff, run)], sem))
        for cp in copies:
            cp.wait()

    return _sc_gather


def kernel(hidden_states, codebook_hidden_states, Wq, bq, Wk, bk, Wv, bv):
    logits, idx1d, valT = _tc_call(
        hidden_states.T, codebook_hidden_states, codebook_hidden_states.T,
        Wq.T, bq.reshape(-1, 1), Wk, bk.reshape(1, -1),
        Wv.T, bv.reshape(-1, 1),
    )
    zq_flat, zq2_flat = _make_sc_gather()(idx1d, valT.reshape(-1))

    def _untile(flat):
        zqT = flat.reshape(4, _NT // 128, 8, 128).transpose(0, 2, 1, 3)
        return zqT.reshape(_C, _NT).T

    return (logits, idx1d.reshape(-1, 1), _untile(zq_flat), _untile(zq2_flat))
